# Initial kernel scaffold; baseline (speedup 1.0000x reference)
import jax
import jax.numpy as jnp
from jax.experimental import pallas as pl


def kernel(input, hidden, char_embed):
    raise NotImplementedError("write your pallas kernel here")


# speedup vs baseline: 6.0687x; 6.0687x over previous
import functools

import jax
import jax.numpy as jnp
from jax import lax
from jax.experimental import pallas as pl
from jax.experimental.pallas import tpu as pltpu
from jax.experimental.pallas import tpu_sc as plsc

_D = 14
_DP = 16
_SUB = 128
_K = 16
_CHUNK = _SUB * _K
_NW = 32


def _make_gather(n_rows: int, vocab: int):
    per_w = n_rows // _NW
    n_blocks = per_w // _CHUNK
    mesh = plsc.VectorSubcoreMesh(core_axis_name="c", subcore_axis_name="s")

    @functools.partial(
        pl.kernel,
        out_type=jax.ShapeDtypeStruct((n_rows, _DP), jnp.float32),
        mesh=mesh,
        scratch_types=[
            pltpu.VMEM((_K, _SUB), jnp.int32),
            pltpu.VMEM((_CHUNK, _DP), jnp.float32),
            pltpu.SemaphoreType.DMA,
        ],
        compiler_params=pltpu.CompilerParams(use_tc_tiling_on_sc=False),
    )
    def gather_kernel(table_hbm, idx_hbm, out_hbm, idx_v, rows_v, sem):
        wid = lax.axis_index("s") * 2 + lax.axis_index("c")
        base = wid * per_w

        def block(b, carry):
            row0 = pl.multiple_of(base + b * _CHUNK, _CHUNK)
            blk0 = pl.multiple_of((base // _SUB) + b * _K, _K)
            pltpu.sync_copy(idx_hbm.at[pl.ds(blk0, _K)], idx_v)
            copies = []
            for j in range(_K):
                copies.append(
                    pltpu.async_copy(
                        table_hbm.at[idx_v.at[j]],
                        rows_v.at[pl.ds(j * _SUB, _SUB)],
                        sem,
                    )
                )
            for c in copies:
                c.wait()
            pltpu.sync_copy(rows_v, out_hbm.at[pl.ds(row0, _CHUNK)])
            return carry

        lax.fori_loop(0, n_blocks, block, 0)

    return gather_kernel


def kernel(input, hidden, char_embed):
    B, L = input.shape[1], input.shape[2]
    n_rows = B * L
    table16 = jnp.pad(char_embed, ((0, 0), (0, _DP - _D)))
    idx2d = input[0].reshape(n_rows // _SUB, _SUB)
    out16 = _make_gather(n_rows, char_embed.shape[0])(table16, idx2d)
    return out16[:, :_D].reshape(B, L, _D)


# speedup vs baseline: 10.1516x; 1.6728x over previous
import functools

import jax
import jax.numpy as jnp
from jax import lax
from jax.experimental import pallas as pl
from jax.experimental.pallas import tpu as pltpu
from jax.experimental.pallas import tpu_sc as plsc

_D = 14
_CH = 4096
_NW = 32


def _make_gather(L: int, B: int, vocab: int):
    half_b = B // 2
    nsub = half_b // _CH
    mesh = plsc.VectorSubcoreMesh(core_axis_name="c", subcore_axis_name="s")

    @functools.partial(
        pl.kernel,
        out_type=jax.ShapeDtypeStruct((_D, L, B), jnp.float32),
        mesh=mesh,
        scratch_types=[
            pltpu.VMEM((vocab,), jnp.float32),
            pltpu.VMEM((_CH,), jnp.int32),
            pltpu.VMEM((_CH,), jnp.float32),
        ],
        compiler_params=pltpu.CompilerParams(
            use_tc_tiling_on_sc=False, needs_layout_passes=False
        ),
    )
    def gather_kernel(table_t_hbm, idxT_hbm, out_hbm, tab_v, idx_v, out_v):
        wid = lax.axis_index("s") * 2 + lax.axis_index("c")
        d = wid // 2
        half = wid % 2

        @pl.when(d < _D)
        def _():
            pltpu.sync_copy(table_t_hbm.at[d], tab_v)

            def tile(t, carry):
                l = t // nsub
                b0 = pl.multiple_of(half * half_b + (t % nsub) * _CH, _CH)
                pltpu.sync_copy(idxT_hbm.at[l, pl.ds(b0, _CH)], idx_v)

                def grp(g, c):
                    o = pl.multiple_of(g * 16, 16)
                    idx16 = idx_v[pl.ds(o, 16)]
                    out_v[pl.ds(o, 16)] = plsc.load_gather(tab_v, [idx16])
                    return c

                lax.fori_loop(0, _CH // 16, grp, 0)
                pltpu.sync_copy(out_v, out_hbm.at[d, l, pl.ds(b0, _CH)])
                return carry

            lax.fori_loop(0, L * nsub, tile, 0)

    return gather_kernel


def kernel(input, hidden, char_embed):
    B, L = input.shape[1], input.shape[2]
    table_t = char_embed.T
    idxT = input[0].T
    out_t = _make_gather(L, B, char_embed.shape[0])(table_t, idxT)
    return jnp.transpose(out_t, (2, 1, 0))


# speedup vs baseline: 18.8772x; 1.8595x over previous
import functools

import jax
import jax.numpy as jnp
from jax import lax
from jax.experimental import pallas as pl
from jax.experimental.pallas import tpu as pltpu
from jax.experimental.pallas import tpu_sc as plsc

_D = 14
_CH = 4096
_UNROLL = 8


def _make_gather(L: int, B: int, vocab: int):
    half_b = B // 2
    nsub = half_b // _CH
    n_tiles = L * nsub
    n_pairs = n_tiles // 2
    mesh = plsc.VectorSubcoreMesh(core_axis_name="c", subcore_axis_name="s")

    @functools.partial(
        pl.kernel,
        out_type=jax.ShapeDtypeStruct((_D, L, B), jnp.float32),
        mesh=mesh,
        scratch_types=[
            pltpu.VMEM((vocab,), jnp.float32),
            pltpu.VMEM((_CH,), jnp.int32),
            pltpu.VMEM((_CH,), jnp.int32),
            pltpu.VMEM((_CH,), jnp.float32),
            pltpu.VMEM((_CH,), jnp.float32),
            pltpu.SemaphoreType.DMA,
            pltpu.SemaphoreType.DMA,
            pltpu.SemaphoreType.DMA,
            pltpu.SemaphoreType.DMA,
        ],
        compiler_params=pltpu.CompilerParams(
            use_tc_tiling_on_sc=False, needs_layout_passes=False
        ),
    )
    def gather_kernel(table_t_hbm, idxT_hbm, out_hbm, tab_v,
                      idx_v0, idx_v1, out_v0, out_v1,
                      isem0, isem1, osem0, osem1):
        wid = lax.axis_index("s") * 2 + lax.axis_index("c")
        d = wid // 2
        half = wid % 2

        def idx_src(t):
            l = t // nsub
            b0 = pl.multiple_of(half * half_b + (t % nsub) * _CH, _CH)
            return idxT_hbm.at[l, pl.ds(b0, _CH)]

        def out_dst(t):
            l = t // nsub
            b0 = pl.multiple_of(half * half_b + (t % nsub) * _CH, _CH)
            return out_hbm.at[d, l, pl.ds(b0, _CH)]

        def compute(idx_v, out_v):
            def grp(g, c):
                o0 = pl.multiple_of(g * 16 * _UNROLL, 16)
                for u in range(_UNROLL):
                    o = o0 + u * 16
                    idx16 = idx_v[pl.ds(o, 16)]
                    out_v[pl.ds(o, 16)] = plsc.load_gather(tab_v, [idx16])
                return c

            lax.fori_loop(0, _CH // (16 * _UNROLL), grp, 0)

        @pl.when(d < _D)
        def _():
            pltpu.sync_copy(table_t_hbm.at[d], tab_v)
            pltpu.async_copy(idx_src(0), idx_v0, isem0)
            pltpu.async_copy(idx_src(1), idx_v1, isem1)

            def pair(p, carry):
                t0 = p * 2
                t1 = t0 + 1
                pltpu.make_async_copy(idx_src(t0), idx_v0, isem0).wait()

                @pl.when(p > 0)
                def _():
                    pltpu.make_async_copy(out_v0, out_dst(t0), osem0).wait()

                compute(idx_v0, out_v0)
                pltpu.async_copy(out_v0, out_dst(t0), osem0)

                @pl.when(p + 1 < n_pairs)
                def _():
                    pltpu.async_copy(idx_src(t0 + 2), idx_v0, isem0)

                pltpu.make_async_copy(idx_src(t1), idx_v1, isem1).wait()

                @pl.when(p > 0)
                def _():
                    pltpu.make_async_copy(out_v1, out_dst(t1), osem1).wait()

                compute(idx_v1, out_v1)
                pltpu.async_copy(out_v1, out_dst(t1), osem1)

                @pl.when(p + 1 < n_pairs)
                def _():
                    pltpu.async_copy(idx_src(t1 + 2), idx_v1, isem1)

                return carry

            lax.fori_loop(0, n_pairs, pair, 0)
            pltpu.make_async_copy(out_v0, out_dst(n_tiles - 2), osem0).wait()
            pltpu.make_async_copy(out_v1, out_dst(n_tiles - 1), osem1).wait()

    return gather_kernel


def kernel(input, hidden, char_embed):
    B, L = input.shape[1], input.shape[2]
    table_t = char_embed.T
    idxT = input[0].T
    out_t = _make_gather(L, B, char_embed.shape[0])(table_t, idxT)
    return jnp.transpose(out_t, (2, 1, 0))


# speedup vs baseline: 25.1273x; 1.3311x over previous
import functools

import jax
import jax.numpy as jnp
from jax import lax
from jax.experimental import pallas as pl
from jax.experimental.pallas import tpu as pltpu
from jax.experimental.pallas import tpu_sc as plsc

_D = 14
_CH = 4096
_UNROLL = 8


def _make_gather(L: int, B: int, vocab: int):
    half_b = B // 2
    nsub = half_b // _CH
    n_tiles = L * nsub
    n_pairs = n_tiles // 2
    mesh = plsc.VectorSubcoreMesh(core_axis_name="c", subcore_axis_name="s")

    @functools.partial(
        pl.kernel,
        out_type=jax.ShapeDtypeStruct((_D, L, B), jnp.float32),
        mesh=mesh,
        scratch_types=[
            pltpu.VMEM((vocab,), jnp.float32),
            pltpu.VMEM((_CH,), jnp.int32),
            pltpu.VMEM((_CH,), jnp.int32),
            pltpu.VMEM((_CH,), jnp.float32),
            pltpu.VMEM((_CH,), jnp.float32),
            pltpu.SemaphoreType.DMA,
            pltpu.SemaphoreType.DMA,
            pltpu.SemaphoreType.DMA,
            pltpu.SemaphoreType.DMA,
        ],
        compiler_params=pltpu.CompilerParams(
            use_tc_tiling_on_sc=False, needs_layout_passes=False
        ),
    )
    def gather_kernel(table_t_hbm, idxT_hbm, out_hbm, tab_v,
                      idx_v0, idx_v1, out_v0, out_v1,
                      isem0, isem1, osem0, osem1):
        wid = lax.axis_index("s") * 2 + lax.axis_index("c")
        d = wid // 2
        half = wid % 2

        def idx_src(t):
            l = t // nsub
            b0 = pl.multiple_of(half * half_b + (t % nsub) * _CH, _CH)
            return idxT_hbm.at[l, pl.ds(b0, _CH)]

        def out_dst(t):
            l = t // nsub
            b0 = pl.multiple_of(half * half_b + (t % nsub) * _CH, _CH)
            return out_hbm.at[d, l, pl.ds(b0, _CH)]

        def compute(idx_v, out_v):
            @plsc.parallel_loop(0, _CH, 16, unroll=_UNROLL)
            def _(g):
                o = pl.multiple_of(g, 16)
                idx16 = idx_v[pl.ds(o, 16)]
                out_v[pl.ds(o, 16)] = plsc.load_gather(tab_v, [idx16])

        @pl.when(d < _D)
        def _():
            pltpu.sync_copy(table_t_hbm.at[d], tab_v)
            pltpu.async_copy(idx_src(0), idx_v0, isem0)
            pltpu.async_copy(idx_src(1), idx_v1, isem1)

            def pair(p, carry):
                t0 = p * 2
                t1 = t0 + 1
                pltpu.make_async_copy(idx_src(t0), idx_v0, isem0).wait()

                @pl.when(p > 0)
                def _():
                    pltpu.make_async_copy(out_v0, out_dst(t0), osem0).wait()

                compute(idx_v0, out_v0)
                pltpu.async_copy(out_v0, out_dst(t0), osem0)

                @pl.when(p + 1 < n_pairs)
                def _():
                    pltpu.async_copy(idx_src(t0 + 2), idx_v0, isem0)

                pltpu.make_async_copy(idx_src(t1), idx_v1, isem1).wait()

                @pl.when(p > 0)
                def _():
                    pltpu.make_async_copy(out_v1, out_dst(t1), osem1).wait()

                compute(idx_v1, out_v1)
                pltpu.async_copy(out_v1, out_dst(t1), osem1)

                @pl.when(p + 1 < n_pairs)
                def _():
                    pltpu.async_copy(idx_src(t1 + 2), idx_v1, isem1)

                return carry

            lax.fori_loop(0, n_pairs, pair, 0)
            pltpu.make_async_copy(out_v0, out_dst(n_tiles - 2), osem0).wait()
            pltpu.make_async_copy(out_v1, out_dst(n_tiles - 1), osem1).wait()

    return gather_kernel


def kernel(input, hidden, char_embed):
    B, L = input.shape[1], input.shape[2]
    table_t = char_embed.T
    idxT = input[0].T
    out_t = _make_gather(L, B, char_embed.shape[0])(table_t, idxT)
    return jnp.transpose(out_t, (2, 1, 0))


# speedup vs baseline: 25.1372x; 1.0004x over previous
import functools

import jax
import jax.numpy as jnp
from jax import lax
from jax.experimental import pallas as pl
from jax.experimental.pallas import tpu as pltpu
from jax.experimental.pallas import tpu_sc as plsc

_D = 14
_CH = 4096
_UNROLL = 16


def _make_gather(L: int, B: int, vocab: int):
    half_b = B // 2
    nsub = half_b // _CH
    n_tiles = L * nsub
    n_pairs = n_tiles // 2
    mesh = plsc.VectorSubcoreMesh(core_axis_name="c", subcore_axis_name="s")

    @functools.partial(
        pl.kernel,
        out_type=jax.ShapeDtypeStruct((_D, L, B), jnp.float32),
        mesh=mesh,
        scratch_types=[
            pltpu.VMEM((vocab,), jnp.float32),
            pltpu.VMEM((_CH,), jnp.int32),
            pltpu.VMEM((_CH,), jnp.int32),
            pltpu.VMEM((_CH,), jnp.float32),
            pltpu.VMEM((_CH,), jnp.float32),
            pltpu.SemaphoreType.DMA,
            pltpu.SemaphoreType.DMA,
            pltpu.SemaphoreType.DMA,
            pltpu.SemaphoreType.DMA,
        ],
        compiler_params=pltpu.CompilerParams(
            use_tc_tiling_on_sc=False, needs_layout_passes=False
        ),
    )
    def gather_kernel(table_t_hbm, idxT_hbm, out_hbm, tab_v,
                      idx_v0, idx_v1, out_v0, out_v1,
                      isem0, isem1, osem0, osem1):
        wid = lax.axis_index("s") * 2 + lax.axis_index("c")
        d = wid // 2
        half = wid % 2

        def idx_src(t):
            l = t // nsub
            b0 = pl.multiple_of(half * half_b + (t % nsub) * _CH, _CH)
            return idxT_hbm.at[l, pl.ds(b0, _CH)]

        def out_dst(t):
            l = t // nsub
            b0 = pl.multiple_of(half * half_b + (t % nsub) * _CH, _CH)
            return out_hbm.at[d, l, pl.ds(b0, _CH)]

        def compute(idx_v, out_v):
            @plsc.parallel_loop(0, _CH, 16, unroll=_UNROLL)
            def _(g):
                o = pl.multiple_of(g, 16)
                idx16 = idx_v[pl.ds(o, 16)]
                out_v[pl.ds(o, 16)] = plsc.load_gather(tab_v, [idx16])

        @pl.when(d < _D)
        def _():
            pltpu.sync_copy(table_t_hbm.at[d], tab_v)
            pltpu.async_copy(idx_src(0), idx_v0, isem0)
            pltpu.async_copy(idx_src(1), idx_v1, isem1)

            def pair(p, carry):
                t0 = p * 2
                t1 = t0 + 1
                pltpu.make_async_copy(idx_src(t0), idx_v0, isem0).wait()

                @pl.when(p > 0)
                def _():
                    pltpu.make_async_copy(out_v0, out_dst(t0), osem0).wait()

                compute(idx_v0, out_v0)
                pltpu.async_copy(out_v0, out_dst(t0), osem0)

                @pl.when(p + 1 < n_pairs)
                def _():
                    pltpu.async_copy(idx_src(t0 + 2), idx_v0, isem0)

                pltpu.make_async_copy(idx_src(t1), idx_v1, isem1).wait()

                @pl.when(p > 0)
                def _():
                    pltpu.make_async_copy(out_v1, out_dst(t1), osem1).wait()

                compute(idx_v1, out_v1)
                pltpu.async_copy(out_v1, out_dst(t1), osem1)

                @pl.when(p + 1 < n_pairs)
                def _():
                    pltpu.async_copy(idx_src(t1 + 2), idx_v1, isem1)

                return carry

            lax.fori_loop(0, n_pairs, pair, 0)
            pltpu.make_async_copy(out_v0, out_dst(n_tiles - 2), osem0).wait()
            pltpu.make_async_copy(out_v1, out_dst(n_tiles - 1), osem1).wait()

    return gather_kernel


def kernel(input, hidden, char_embed):
    B, L = input.shape[1], input.shape[2]
    table_t = char_embed.T
    idxT = input[0].T
    out_t = _make_gather(L, B, char_embed.shape[0])(table_t, idxT)
    return jnp.transpose(out_t, (2, 1, 0))
